# Initial kernel scaffold; baseline (speedup 1.0000x reference)
import jax
import jax.numpy as jnp
from jax.experimental import pallas as pl


def kernel(x, adj_t, z, z_table, W1, b1, g1, be1, W2, b2, Wmu, bmu):
    raise NotImplementedError("write your pallas kernel here")


# speedup vs baseline: 8.7783x; 8.7783x over previous
import functools

import jax
import jax.numpy as jnp
from jax import lax
from jax.experimental import pallas as pl
from jax.experimental.pallas import tpu as pltpu
from jax.experimental.pallas import tpu_sc as plsc

NC, NS, LANES = 2, 16, 16
N = 10000
N_PAD = 10240
CH = 128
E_PAD_UNIT = NC * NS * CH

BR = 512
GRID = N_PAD // BR
HIGHEST = jax.lax.Precision.HIGHEST


def _sc_degree(dst_pad, e_pad):
    per_core = e_pad // NC
    per_tile = per_core // NS
    chunks = per_tile // CH
    rpt = N_PAD // NS

    mesh = plsc.VectorSubcoreMesh(core_axis_name="c", subcore_axis_name="s")

    @functools.partial(
        pl.kernel,
        out_type=jax.ShapeDtypeStruct((NC, N_PAD, LANES), jnp.float32),
        mesh=mesh,
        scratch_types=[
            pltpu.VMEM((CH,), jnp.int32),
            pltpu.VMEM((CH, LANES), jnp.float32),
            pltpu.VMEM_SHARED((N_PAD, LANES), jnp.float32),
            pltpu.SemaphoreType.DMA,
        ],
    )
    def k(dst_hbm, out_hbm, di, ones_v, acc_sh, sem):
        cid = lax.axis_index("c")
        sid = lax.axis_index("s")

        @pl.loop(0, CH)
        def _(r):
            ones_v[r, :] = jnp.zeros((LANES,), jnp.float32)

        @pl.loop(0, rpt, step=CH)
        def _(off):
            pltpu.sync_copy(ones_v, acc_sh.at[pl.ds(sid * rpt + off, CH)])

        @pl.loop(0, CH)
        def _(r):
            ones_v[r, :] = jnp.ones((LANES,), jnp.float32)

        plsc.subcore_barrier()

        base0 = cid * per_core + sid * per_tile

        @pl.loop(0, chunks)
        def _(j):
            pltpu.sync_copy(dst_hbm.at[pl.ds(base0 + j * CH, CH)], di)
            pltpu.sync_copy(ones_v, acc_sh.at[di], add=True)

        plsc.subcore_barrier()
        pltpu.sync_copy(acc_sh.at[pl.ds(sid * rpt, rpt)],
                        out_hbm.at[cid, pl.ds(sid * rpt, rpt)])

    return k(dst_pad)


def _sc_aggregate(hs, src_pad, dst_pad, e_pad, dw):
    per_core = e_pad // NC
    per_tile = per_core // NS
    chunks = per_tile // CH
    rpt = N_PAD // NS

    mesh = plsc.VectorSubcoreMesh(core_axis_name="c", subcore_axis_name="s")

    @functools.partial(
        pl.kernel,
        out_type=jax.ShapeDtypeStruct((NC, N_PAD, dw), jnp.float32),
        mesh=mesh,
        scratch_types=[
            pltpu.VMEM((CH,), jnp.int32),
            pltpu.VMEM((CH,), jnp.int32),
            pltpu.VMEM((CH, dw), jnp.float32),
            pltpu.VMEM_SHARED((N_PAD, dw), jnp.float32),
            pltpu.SemaphoreType.DMA,
        ],
    )
    def k(hs_hbm, src_hbm, dst_hbm, out_hbm, si, di, rows, acc_sh, sem):
        cid = lax.axis_index("c")
        sid = lax.axis_index("s")

        @pl.loop(0, CH)
        def _(r):
            @pl.loop(0, dw, step=LANES)
            def _(c):
                rows[r, pl.ds(c, LANES)] = jnp.zeros((LANES,), jnp.float32)

        @pl.loop(0, rpt, step=CH)
        def _(off):
            pltpu.sync_copy(rows, acc_sh.at[pl.ds(sid * rpt + off, CH)])

        plsc.subcore_barrier()

        base0 = cid * per_core + sid * per_tile

        @pl.loop(0, chunks)
        def _(j):
            base = base0 + j * CH
            pltpu.sync_copy(src_hbm.at[pl.ds(base, CH)], si)
            pltpu.sync_copy(dst_hbm.at[pl.ds(base, CH)], di)
            pltpu.async_copy(hs_hbm.at[si], rows, sem).wait()
            pltpu.sync_copy(rows, acc_sh.at[di], add=True)

        plsc.subcore_barrier()
        pltpu.sync_copy(acc_sh.at[pl.ds(sid * rpt, rpt)],
                        out_hbm.at[cid, pl.ds(sid * rpt, rpt)])

    return k(hs, src_pad, dst_pad)


def _tc_lin1(x_pad, zf, z_table, W1):
    d_feat = x_pad.shape[1]
    hid = W1.shape[1]

    def body(x_ref, zf_ref, zt_ref, w_ref, o_ref):
        w = w_ref[...]
        t = jnp.dot(zt_ref[...], w[d_feat:, :], precision=HIGHEST,
                    preferred_element_type=jnp.float32)
        acc = jnp.dot(x_ref[...], w[:d_feat, :], precision=HIGHEST,
                      preferred_element_type=jnp.float32)
        zfb = zf_ref[...]
        o_ref[...] = acc + t[0:1, :] + zfb * (t[1:2, :] - t[0:1, :])

    return pl.pallas_call(
        body,
        grid=(GRID,),
        in_specs=[
            pl.BlockSpec((BR, d_feat), lambda i: (i, 0)),
            pl.BlockSpec((BR, 1), lambda i: (i, 0)),
            pl.BlockSpec(z_table.shape, lambda i: (0, 0)),
            pl.BlockSpec(W1.shape, lambda i: (0, 0)),
        ],
        out_specs=pl.BlockSpec((BR, hid), lambda i: (i, 0)),
        out_shape=jax.ShapeDtypeStruct((N_PAD, hid), jnp.float32),
    )(x_pad, zf, z_table, W1)


def _tc_scale(degp, h1lin):
    hid = h1lin.shape[1]

    def body(deg_ref, h_ref, dinv_ref, hs_ref):
        indeg = (deg_ref[0] + deg_ref[1])[:, 0:1]
        dinv = lax.rsqrt(indeg + 1.0)
        dinv_ref[...] = dinv
        hs_ref[...] = dinv * h_ref[...]

    return pl.pallas_call(
        body,
        grid=(GRID,),
        in_specs=[
            pl.BlockSpec((NC, BR, LANES), lambda i: (0, i, 0)),
            pl.BlockSpec((BR, hid), lambda i: (i, 0)),
        ],
        out_specs=[
            pl.BlockSpec((BR, 1), lambda i: (i, 0)),
            pl.BlockSpec((BR, hid), lambda i: (i, 0)),
        ],
        out_shape=[
            jax.ShapeDtypeStruct((N_PAD, 1), jnp.float32),
            jax.ShapeDtypeStruct((N_PAD, hid), jnp.float32),
        ],
    )(degp, h1lin)


def _tc_layer2(parts, hs1, dinv, b1, g1, be1, W2):
    hid = hs1.shape[1]
    dout = W2.shape[1]

    def body(p_ref, hs_ref, dinv_ref, b_ref, g_ref, be_ref, w_ref, o_ref):
        dinv = dinv_ref[...]
        c = dinv * (p_ref[0] + p_ref[1] + hs_ref[...]) + b_ref[...]
        mu = jnp.mean(c, axis=-1, keepdims=True)
        xc = c - mu
        var = jnp.mean(xc * xc, axis=-1, keepdims=True)
        u = xc * lax.rsqrt(var + 1e-5) * g_ref[...] + be_ref[...]
        u = jnp.maximum(u, 0.0)
        h2 = jnp.dot(u, w_ref[...], precision=HIGHEST,
                     preferred_element_type=jnp.float32)
        o_ref[...] = dinv * h2

    return pl.pallas_call(
        body,
        grid=(GRID,),
        in_specs=[
            pl.BlockSpec((NC, BR, hid), lambda i: (0, i, 0)),
            pl.BlockSpec((BR, hid), lambda i: (i, 0)),
            pl.BlockSpec((BR, 1), lambda i: (i, 0)),
            pl.BlockSpec((1, hid), lambda i: (0, 0)),
            pl.BlockSpec((1, hid), lambda i: (0, 0)),
            pl.BlockSpec((1, hid), lambda i: (0, 0)),
            pl.BlockSpec(W2.shape, lambda i: (0, 0)),
        ],
        out_specs=pl.BlockSpec((BR, dout), lambda i: (i, 0)),
        out_shape=jax.ShapeDtypeStruct((N_PAD, dout), jnp.float32),
    )(parts, hs1, dinv, b1, g1, be1, W2)


def _tc_layer3(parts, hs2, dinv, b2, Wmu):
    hid = hs2.shape[1]
    dout = Wmu.shape[1]

    def body(p_ref, hs_ref, dinv_ref, b_ref, w_ref, o_ref):
        dinv = dinv_ref[...]
        c = dinv * (p_ref[0] + p_ref[1] + hs_ref[...]) + b_ref[...]
        u = jnp.maximum(c, 0.0)
        h3 = jnp.dot(u, w_ref[...], precision=HIGHEST,
                     preferred_element_type=jnp.float32)
        o_ref[...] = dinv * h3

    return pl.pallas_call(
        body,
        grid=(GRID,),
        in_specs=[
            pl.BlockSpec((NC, BR, hid), lambda i: (0, i, 0)),
            pl.BlockSpec((BR, hid), lambda i: (i, 0)),
            pl.BlockSpec((BR, 1), lambda i: (i, 0)),
            pl.BlockSpec((1, hid), lambda i: (0, 0)),
            pl.BlockSpec(Wmu.shape, lambda i: (0, 0)),
        ],
        out_specs=pl.BlockSpec((BR, dout), lambda i: (i, 0)),
        out_shape=jax.ShapeDtypeStruct((N_PAD, dout), jnp.float32),
    )(parts, hs2, dinv, b2, Wmu)


def _tc_final(parts, hs3, dinv, bmu):
    wide = hs3.shape[1]
    dout = bmu.shape[1]

    def body(p_ref, hs_ref, dinv_ref, b_ref, o_ref):
        c = dinv_ref[...] * (p_ref[0] + p_ref[1] + hs_ref[...])
        o_ref[...] = c[:, :dout] + b_ref[...]

    return pl.pallas_call(
        body,
        grid=(GRID,),
        in_specs=[
            pl.BlockSpec((NC, BR, wide), lambda i: (0, i, 0)),
            pl.BlockSpec((BR, wide), lambda i: (i, 0)),
            pl.BlockSpec((BR, 1), lambda i: (i, 0)),
            pl.BlockSpec((1, dout), lambda i: (0, 0)),
        ],
        out_specs=pl.BlockSpec((BR, dout), lambda i: (i, 0)),
        out_shape=jax.ShapeDtypeStruct((N_PAD, dout), jnp.float32),
    )(parts, hs3, dinv, bmu)


def kernel(x, adj_t, z, z_table, W1, b1, g1, be1, W2, b2, Wmu, bmu):
    n, d_feat = x.shape
    e = adj_t.shape[1]
    e_pad = ((e + E_PAD_UNIT - 1) // E_PAD_UNIT) * E_PAD_UNIT

    src = adj_t[0].astype(jnp.int32)
    dst = adj_t[1].astype(jnp.int32)
    src_pad = jnp.concatenate([src, jnp.full((e_pad - e,), n, jnp.int32)])
    dst_pad = jnp.concatenate([dst, jnp.full((e_pad - e,), n, jnp.int32)])

    x_pad = jnp.pad(x, ((0, N_PAD - n), (0, 0)))
    zf = jnp.pad(z.astype(jnp.float32), (0, N_PAD - n)).reshape(N_PAD, 1)

    b1r = b1.reshape(1, -1)
    g1r = g1.reshape(1, -1)
    be1r = be1.reshape(1, -1)
    b2r = b2.reshape(1, -1)
    bmur = bmu.reshape(1, -1)

    degp = _sc_degree(dst_pad, e_pad)
    h1lin = _tc_lin1(x_pad, zf, z_table, W1)
    dinv, hs1 = _tc_scale(degp, h1lin)

    agg1 = _sc_aggregate(hs1, src_pad, dst_pad, e_pad, hs1.shape[1])
    hs2 = _tc_layer2(agg1, hs1, dinv, b1r, g1r, be1r, W2)

    wmu_pad = jnp.pad(Wmu, ((0, 0), (0, W2.shape[1] - Wmu.shape[1])))

    agg2 = _sc_aggregate(hs2, src_pad, dst_pad, e_pad, hs2.shape[1])
    hs3 = _tc_layer3(agg2, hs2, dinv, b2r, wmu_pad)

    agg3 = _sc_aggregate(hs3, src_pad, dst_pad, e_pad, hs3.shape[1])
    out = _tc_final(agg3, hs3, dinv, bmur)

    return out[:n]
